# baseline (speedup 1.0000x reference)
import functools

import jax
import jax.numpy as jnp
from jax import lax
from jax.experimental import pallas as pl
from jax.experimental.pallas import tpu as pltpu
from jax.experimental.pallas import tpu_sc as plsc

_NAT = 40
_NBOND = 4
_NBT = 7
_BN_F = 64
_K_F = 32
_N_BLOCKS = 1024
_CHUNK = 1000


def _elu(x):
    return jnp.where(x > 0, x, jnp.exp(jnp.minimum(x, 0.0)) - 1.0)


_N_EDGE = 800000
_N_KEY = 200000
_EPT_PAD = 25600
_ECH = 3200
_NECH = 8
_CBUF = _ECH + 512
_PASS_KEYS = 16384
_N_PASS = 13
_KEY_PAD = _PASS_KEYS * _N_PASS
_CH = 512
_ZCH = 128
_ACC_ROWS = _PASS_KEYS + 16


def _edge_aggregate(h, ekey_pad, eend_pad):
    i32 = jnp.int32
    f32 = jnp.float32
    mesh = plsc.VectorSubcoreMesh(core_axis_name="c", subcore_axis_name="s")

    @functools.partial(
        pl.kernel,
        mesh=mesh,
        compiler_params=pltpu.CompilerParams(
            needs_layout_passes=False, use_tc_tiling_on_sc=False),
        out_type=jax.ShapeDtypeStruct((2, _KEY_PAD, _BN_F), f32),
        scratch_types=[
            pltpu.VMEM((_ECH,), i32),
            pltpu.VMEM((_ECH,), i32),
            pltpu.VMEM((_CBUF,), i32),
            pltpu.VMEM((_CBUF,), i32),
            pltpu.VMEM((_CH, _BN_F), f32),
            pltpu.VMEM((_ZCH, _BN_F), f32),
            pltpu.VMEM_SHARED((_ACC_ROWS, _BN_F), f32),
            pltpu.SemaphoreType.DMA,
        ],
    )
    def agg_kernel(h_hbm, ekey_hbm, eend_hbm, out_hbm, vkeys, vends, cend,
                   clkey, staging, zerobuf, accum, sem):
        c = lax.axis_index("c")
        s = lax.axis_index("s")
        wid = c * 16 + s
        lane = lax.iota(i32, 16)
        ebase = pl.multiple_of(wid * _EPT_PAD, 8)
        rows = _PASS_KEYS // 16
        rbase = pl.multiple_of(s * rows, 8)

        zv = jnp.zeros((16,), f32)
        for r in range(_ZCH):
            for k in range(_BN_F // 16):
                zerobuf[r, pl.ds(k * 16, 16)] = zv

        def edge_chunk(ec, lo):
            eoff = pl.multiple_of(ebase + ec * _ECH, 8)
            pltpu.sync_copy(ekey_hbm.at[pl.ds(eoff, _ECH)], vkeys)
            pltpu.sync_copy(eend_hbm.at[pl.ds(eoff, _ECH)], vends)

            def compact(i, cur):
                off = pl.multiple_of(i * 32, 32)
                kv1 = vkeys[pl.ds(off, 16)]
                ev1 = vends[pl.ds(off, 16)]
                kv2 = vkeys[pl.ds(off + 16, 16)]
                ev2 = vends[pl.ds(off + 16, 16)]
                m1 = (kv1 >= lo) & (kv1 < lo + _PASS_KEYS)
                m2 = (kv2 >= lo) & (kv2 < lo + _PASS_KEYS)
                mi1 = m1.astype(i32)
                mi2 = m2.astype(i32)
                pc1 = plsc.all_reduce_population_count(m1)
                pc2 = plsc.all_reduce_population_count(m2)
                pos1 = cur + plsc.cumsum(mi1) - mi1
                cur2 = cur + pc1
                pos2 = cur2 + plsc.cumsum(mi2) - mi2
                plsc.store_scatter(cend, [pos1], ev1, mask=m1)
                plsc.store_scatter(clkey, [pos1], kv1 - lo, mask=m1)
                plsc.store_scatter(cend, [pos2], ev2, mask=m2)
                plsc.store_scatter(clkey, [pos2], kv2 - lo, mask=m2)
                return cur2 + pc2

            cnt_v = lax.fori_loop(0, _ECH // 32, compact,
                                  jnp.zeros((16,), i32))
            cnt = cnt_v[0]
            for t in range(_CH // 16):
                tpos = cnt + t * 16 + lane
                plsc.store_scatter(cend, [tpos], jnp.zeros((16,), i32))
                plsc.store_scatter(clkey, [tpos],
                                   jnp.full((16,), _PASS_KEYS, i32))

            def row_chunk(j, carry):
                jb = pl.multiple_of(j * _CH, _CH)
                pltpu.async_copy(h_hbm.at[cend.at[pl.ds(jb, _CH)]],
                                 staging, sem).wait()
                pltpu.sync_copy(staging, accum.at[clkey.at[pl.ds(jb, _CH)]],
                                add=True)
                return carry

            lax.fori_loop(0, (cnt + _CH - 1) // _CH, row_chunk, 0)
            return lo

        def one_pass(p, carry):
            lo = p * _PASS_KEYS
            for q in range(rows // _ZCH):
                pltpu.sync_copy(zerobuf, accum.at[pl.ds(rbase + q * _ZCH, _ZCH)])
            plsc.subcore_barrier()
            lax.fori_loop(0, _NECH, edge_chunk, lo)
            plsc.subcore_barrier()
            pltpu.sync_copy(
                accum.at[pl.ds(rbase, rows)],
                out_hbm.at[c, pl.ds(pl.multiple_of(lo + s * rows, 8), rows)])
            return carry

        lax.fori_loop(0, _N_PASS, one_pass, 0)

    return agg_kernel(h, ekey_pad, eend_pad)


def _mlp3_agg(h, p0, p1, W1h, W1a, b1, W2, b2, W3, b3):
    N = h.shape[0]
    Dh = h.shape[1]
    Da = W1a.shape[0]
    H1 = W1h.shape[1]
    H2 = W2.shape[1]
    F = W3.shape[1]
    grid = N // _CHUNK

    def body(h_ref, p0_ref, p1_ref, w1h_ref, w1a_ref, b1_ref, w2_ref, b2_ref,
             w3_ref, b3_ref, o_ref):
        a = p0_ref[...] + p1_ref[...]
        t = (jnp.dot(h_ref[...], w1h_ref[...], preferred_element_type=jnp.float32)
             + jnp.dot(a, w1a_ref[...], preferred_element_type=jnp.float32)
             + b1_ref[...])
        t = _elu(t)
        t = _elu(jnp.dot(t, w2_ref[...], preferred_element_type=jnp.float32)
                 + b2_ref[...])
        o_ref[...] = (jnp.dot(t, w3_ref[...], preferred_element_type=jnp.float32)
                      + b3_ref[...])

    const = lambda i: (0, 0)
    return pl.pallas_call(
        body,
        grid=(grid,),
        in_specs=[
            pl.BlockSpec((_CHUNK, Dh), lambda i: (i, 0)),
            pl.BlockSpec((_CHUNK, Da), lambda i: (i, 0)),
            pl.BlockSpec((_CHUNK, Da), lambda i: (i, 0)),
            pl.BlockSpec((Dh, H1), const), pl.BlockSpec((Da, H1), const),
            pl.BlockSpec((1, H1), const),
            pl.BlockSpec((H1, H2), const), pl.BlockSpec((1, H2), const),
            pl.BlockSpec((H2, F), const), pl.BlockSpec((1, F), const),
        ],
        out_specs=pl.BlockSpec((_CHUNK, F), lambda i: (i, 0)),
        out_shape=jax.ShapeDtypeStruct((N, F), jnp.float32),
    )(h, p0, p1, W1h, W1a, b1.reshape(1, H1), W2, b2.reshape(1, H2), W3,
      b3.reshape(1, F))


def _bnl_matmul(x, gamma, beta, W, b, apply_elu_pre=True):
    N, D = x.shape
    F = W.shape[1]
    grid = N // _CHUNK

    def body(x_ref, g_ref, be_ref, w_ref, b_ref, o_ref):
        a = x_ref[...]
        if apply_elu_pre:
            a = _elu(a * g_ref[...] + be_ref[...])
        o_ref[...] = (jnp.dot(a, w_ref[...], preferred_element_type=jnp.float32)
                      + b_ref[...])

    return pl.pallas_call(
        body,
        grid=(grid,),
        in_specs=[
            pl.BlockSpec((_CHUNK, D), lambda i: (i, 0)),
            pl.BlockSpec((1, D), lambda i: (0, 0)),
            pl.BlockSpec((1, D), lambda i: (0, 0)),
            pl.BlockSpec((D, F), lambda i: (0, 0)),
            pl.BlockSpec((1, F), lambda i: (0, 0)),
        ],
        out_specs=pl.BlockSpec((_CHUNK, F), lambda i: (i, 0)),
        out_shape=jax.ShapeDtypeStruct((N, F), jnp.float32),
    )(x, gamma.reshape(1, D), beta.reshape(1, D), W, b.reshape(1, F))


def _mlp3(z, W1, b1, W2, b2, W3, b3):
    N, D = z.shape
    H1 = W1.shape[1]
    H2 = W2.shape[1]
    F = W3.shape[1]
    grid = N // _CHUNK

    def body(z_ref, w1_ref, b1_ref, w2_ref, b2_ref, w3_ref, b3_ref, o_ref):
        t = _elu(jnp.dot(z_ref[...], w1_ref[...],
                         preferred_element_type=jnp.float32) + b1_ref[...])
        t = _elu(jnp.dot(t, w2_ref[...],
                         preferred_element_type=jnp.float32) + b2_ref[...])
        o_ref[...] = (jnp.dot(t, w3_ref[...],
                              preferred_element_type=jnp.float32) + b3_ref[...])

    const = lambda i: (0, 0)
    return pl.pallas_call(
        body,
        grid=(grid,),
        in_specs=[
            pl.BlockSpec((_CHUNK, D), lambda i: (i, 0)),
            pl.BlockSpec((D, H1), const), pl.BlockSpec((1, H1), const),
            pl.BlockSpec((H1, H2), const), pl.BlockSpec((1, H2), const),
            pl.BlockSpec((H2, F), const), pl.BlockSpec((1, F), const),
        ],
        out_specs=pl.BlockSpec((_CHUNK, F), lambda i: (i, 0)),
        out_shape=jax.ShapeDtypeStruct((N, F), jnp.float32),
    )(z, W1, b1.reshape(1, H1), W2, b2.reshape(1, H2), W3, b3.reshape(1, F))


def kernel(params, atom_types, is_scaffold, bond_info, block_ids, last_append_mask):
    n = atom_types.shape[0]
    begin = bond_info[:, 0]
    end = bond_info[:, 1]
    btype = bond_info[:, 2]

    at = jnp.where(is_scaffold == 1, atom_types + _NAT,
         jnp.where(last_append_mask == 1, atom_types + 2 * _NAT,
         jnp.where(last_append_mask == 2, atom_types + 3 * _NAT, atom_types)))
    at = jnp.where(is_scaffold == 1, at + _NAT, at)
    feats = jnp.take(params['emb'], at, axis=0)

    pad_e = _EPT_PAD - 25000
    ekey = jnp.pad((begin * _NBOND + btype).reshape(32, 25000),
                   ((0, 0), (0, pad_e)), constant_values=2**28).reshape(-1)
    eend = jnp.pad(end.reshape(32, 25000),
                   ((0, 0), (0, pad_e))).reshape(-1)

    for lp in params['layers']:
        bn = lp['bn']
        h = _bnl_matmul(feats, bn['gamma'], bn['beta'], bn['W'], bn['b'])
        parts = _edge_aggregate(h, ekey, eend)
        p0 = parts[0].reshape(_KEY_PAD // _NBOND, _NBOND * _BN_F)
        p1 = parts[1].reshape(_KEY_PAD // _NBOND, _NBOND * _BN_F)
        mlp = lp['mlp']
        W1 = mlp[0]['W']
        z = _mlp3_agg(h, p0, p1,
                      W1[:_BN_F], W1[_BN_F:_BN_F * (1 + _NBOND)], mlp[0]['b'],
                      mlp[1]['W'], mlp[1]['b'], mlp[2]['W'], mlp[2]['b'])
        feats = jnp.concatenate([feats, z], axis=-1)

    fin = params['final']
    out = _bnl_matmul(feats, fin['gamma'], fin['beta'], fin['W'], fin['b'])
    hp = _elu(out * params['pool_gamma'] + params['pool_beta'])

    seg_sum = jax.ops.segment_sum(hp, block_ids, num_segments=_N_BLOCKS)
    cnt = jax.ops.segment_sum(jnp.ones((n,), jnp.float32), block_ids,
                              num_segments=_N_BLOCKS)
    mol = seg_sum / jnp.maximum(cnt, 1.0)[:, None]

    ac = params['append_connect']
    D1 = out.shape[1]
    U = _bnl_matmul(out, ac['gamma'][:D1], ac['beta'][:D1], ac['W'][:D1], ac['b'])
    Vsmall = _elu(mol * ac['gamma'][D1:] + ac['beta'][D1:]) @ ac['W'][D1:]
    act_ac = U + jnp.take(Vsmall, block_ids, axis=0)

    ep = params['end']
    act_end = (_elu(mol * ep['gamma'] + ep['beta']) @ ep['W'] + ep['b'])[:, 0]

    row_max = jnp.max(act_ac, axis=-1)
    seg_max = jax.ops.segment_max(row_max, block_ids, num_segments=_N_BLOCKS)
    m = jnp.maximum(seg_max, act_end)
    ex = jnp.exp(act_ac - jnp.take(m, block_ids)[:, None])
    eb = jnp.exp(act_end - m)
    Z = jax.ops.segment_sum(jnp.sum(ex, axis=-1), block_ids,
                            num_segments=_N_BLOCKS) + eb
    p_ac = ex / jnp.take(Z, block_ids)[:, None]
    p_end = eb / Z
    p_append = p_ac[:, :_NAT * _NBOND].reshape(n, _NAT, _NBOND)
    p_connect = p_ac[:, _NAT * _NBOND:]
    return (p_append, p_connect, p_end)


# speedup vs baseline: 10.5806x; 10.5806x over previous
import functools

import jax
import jax.numpy as jnp
from jax import lax
from jax.experimental import pallas as pl
from jax.experimental.pallas import tpu as pltpu
from jax.experimental.pallas import tpu_sc as plsc

_NAT = 40
_NBOND = 4
_NBT = 7
_BN_F = 64
_K_F = 32
_N_BLOCKS = 1024
_CHUNK = 1000


def _elu(x):
    return jnp.where(x > 0, x, jnp.exp(jnp.minimum(x, 0.0)) - 1.0)


_N_EDGE = 800000
_N_KEY = 200000
_EPT_PAD = 25600
_ECH = 3200
_NECH = 8
_CBUF = _ECH + 128
_PASS_KEYS = 16384
_N_PASS = 13
_KEY_PAD = _PASS_KEYS * _N_PASS
_CH = 128
_ACC_ROWS = _PASS_KEYS + 128


def _edge_aggregate(h, ekey_pad, eend_pad):
    i32 = jnp.int32
    f32 = jnp.float32
    mesh = plsc.VectorSubcoreMesh(core_axis_name="c", subcore_axis_name="s")

    @functools.partial(
        pl.kernel,
        mesh=mesh,
        compiler_params=pltpu.CompilerParams(
            needs_layout_passes=False, use_tc_tiling_on_sc=False),
        out_type=jax.ShapeDtypeStruct((2, _KEY_PAD, _BN_F), f32),
        scratch_types=[
            pltpu.VMEM((_ECH,), i32),
            pltpu.VMEM((_ECH,), i32),
            pltpu.VMEM((_CBUF,), i32),
            pltpu.VMEM((_CBUF,), i32),
            pltpu.VMEM((_CH, _BN_F), f32),
            pltpu.VMEM((_CH, _BN_F), f32),
            pltpu.VMEM_SHARED((_ACC_ROWS, _BN_F), f32),
            pltpu.SemaphoreType.DMA,
        ],
    )
    def agg_kernel(h_hbm, ekey_hbm, eend_hbm, out_hbm, vkeys, vends, cend,
                   clkey, staging, zerobuf, accum, sem):
        c = lax.axis_index("c")
        s = lax.axis_index("s")
        wid = c * 16 + s
        lane = lax.iota(i32, 16)
        ebase = pl.multiple_of(wid * _EPT_PAD, 8)
        rows = _PASS_KEYS // 16
        rbase = pl.multiple_of(s * rows, 8)

        zv = jnp.zeros((16,), f32)
        for r in range(_CH):
            for k in range(_BN_F // 16):
                zerobuf[r, pl.ds(k * 16, 16)] = zv

        def edge_chunk(ec, lo):
            eoff = pl.multiple_of(ebase + ec * _ECH, 8)
            pltpu.sync_copy(ekey_hbm.at[pl.ds(eoff, _ECH)], vkeys)
            pltpu.sync_copy(eend_hbm.at[pl.ds(eoff, _ECH)], vends)

            def compact(i, cur):
                off = pl.multiple_of(i * 32, 32)
                kv1 = vkeys[pl.ds(off, 16)]
                ev1 = vends[pl.ds(off, 16)]
                kv2 = vkeys[pl.ds(off + 16, 16)]
                ev2 = vends[pl.ds(off + 16, 16)]
                m1 = (kv1 >= lo) & (kv1 < lo + _PASS_KEYS)
                m2 = (kv2 >= lo) & (kv2 < lo + _PASS_KEYS)
                mi1 = m1.astype(i32)
                mi2 = m2.astype(i32)
                pc1 = plsc.all_reduce_population_count(m1)
                pc2 = plsc.all_reduce_population_count(m2)
                pos1 = cur + plsc.cumsum(mi1) - mi1
                cur2 = cur + pc1
                pos2 = cur2 + plsc.cumsum(mi2) - mi2
                plsc.store_scatter(cend, [pos1], ev1, mask=m1)
                plsc.store_scatter(clkey, [pos1], kv1 - lo, mask=m1)
                plsc.store_scatter(cend, [pos2], ev2, mask=m2)
                plsc.store_scatter(clkey, [pos2], kv2 - lo, mask=m2)
                return cur2 + pc2

            cnt_v = lax.fori_loop(0, _ECH // 32, compact,
                                  jnp.zeros((16,), i32))
            cnt = cnt_v[0]
            for t in range(_CH // 16):
                tpos = cnt + t * 16 + lane
                plsc.store_scatter(cend, [tpos], t * 16 + lane)
                plsc.store_scatter(clkey, [tpos],
                                   _PASS_KEYS + t * 16 + lane)

            def row_chunk(j, carry):
                jb = pl.multiple_of(j * _CH, _CH)
                pltpu.async_copy(h_hbm.at[cend.at[pl.ds(jb, _CH)]],
                                 staging, sem).wait()
                pltpu.sync_copy(staging, accum.at[clkey.at[pl.ds(jb, _CH)]],
                                add=True)
                return carry

            lax.fori_loop(0, (cnt + _CH - 1) // _CH, row_chunk, 0)
            return lo

        def one_pass(p, carry):
            lo = p * _PASS_KEYS
            for q in range(rows // _CH):
                pltpu.sync_copy(zerobuf, accum.at[pl.ds(rbase + q * _CH, _CH)])
            plsc.subcore_barrier()
            lax.fori_loop(0, _NECH, edge_chunk, lo)
            plsc.subcore_barrier()
            pltpu.sync_copy(
                accum.at[pl.ds(rbase, rows)],
                out_hbm.at[c, pl.ds(pl.multiple_of(lo + s * rows, 8), rows)])
            return carry

        lax.fori_loop(0, _N_PASS, one_pass, 0)

    return agg_kernel(h, ekey_pad, eend_pad)


def _mlp3_agg(h, p0, p1, W1h, W1a, b1, W2, b2, W3, b3):
    N = h.shape[0]
    Dh = h.shape[1]
    Da = W1a.shape[0]
    H1 = W1h.shape[1]
    H2 = W2.shape[1]
    F = W3.shape[1]
    grid = N // _CHUNK

    def body(h_ref, p0_ref, p1_ref, w1h_ref, w1a_ref, b1_ref, w2_ref, b2_ref,
             w3_ref, b3_ref, o_ref):
        a = p0_ref[...] + p1_ref[...]
        t = (jnp.dot(h_ref[...], w1h_ref[...], preferred_element_type=jnp.float32)
             + jnp.dot(a, w1a_ref[...], preferred_element_type=jnp.float32)
             + b1_ref[...])
        t = _elu(t)
        t = _elu(jnp.dot(t, w2_ref[...], preferred_element_type=jnp.float32)
                 + b2_ref[...])
        o_ref[...] = (jnp.dot(t, w3_ref[...], preferred_element_type=jnp.float32)
                      + b3_ref[...])

    const = lambda i: (0, 0)
    return pl.pallas_call(
        body,
        grid=(grid,),
        in_specs=[
            pl.BlockSpec((_CHUNK, Dh), lambda i: (i, 0)),
            pl.BlockSpec((_CHUNK, Da), lambda i: (i, 0)),
            pl.BlockSpec((_CHUNK, Da), lambda i: (i, 0)),
            pl.BlockSpec((Dh, H1), const), pl.BlockSpec((Da, H1), const),
            pl.BlockSpec((1, H1), const),
            pl.BlockSpec((H1, H2), const), pl.BlockSpec((1, H2), const),
            pl.BlockSpec((H2, F), const), pl.BlockSpec((1, F), const),
        ],
        out_specs=pl.BlockSpec((_CHUNK, F), lambda i: (i, 0)),
        out_shape=jax.ShapeDtypeStruct((N, F), jnp.float32),
    )(h, p0, p1, W1h, W1a, b1.reshape(1, H1), W2, b2.reshape(1, H2), W3,
      b3.reshape(1, F))


def _bnl_matmul(x, gamma, beta, W, b, apply_elu_pre=True):
    N, D = x.shape
    F = W.shape[1]
    grid = N // _CHUNK

    def body(x_ref, g_ref, be_ref, w_ref, b_ref, o_ref):
        a = x_ref[...]
        if apply_elu_pre:
            a = _elu(a * g_ref[...] + be_ref[...])
        o_ref[...] = (jnp.dot(a, w_ref[...], preferred_element_type=jnp.float32)
                      + b_ref[...])

    return pl.pallas_call(
        body,
        grid=(grid,),
        in_specs=[
            pl.BlockSpec((_CHUNK, D), lambda i: (i, 0)),
            pl.BlockSpec((1, D), lambda i: (0, 0)),
            pl.BlockSpec((1, D), lambda i: (0, 0)),
            pl.BlockSpec((D, F), lambda i: (0, 0)),
            pl.BlockSpec((1, F), lambda i: (0, 0)),
        ],
        out_specs=pl.BlockSpec((_CHUNK, F), lambda i: (i, 0)),
        out_shape=jax.ShapeDtypeStruct((N, F), jnp.float32),
    )(x, gamma.reshape(1, D), beta.reshape(1, D), W, b.reshape(1, F))


def _mlp3(z, W1, b1, W2, b2, W3, b3):
    N, D = z.shape
    H1 = W1.shape[1]
    H2 = W2.shape[1]
    F = W3.shape[1]
    grid = N // _CHUNK

    def body(z_ref, w1_ref, b1_ref, w2_ref, b2_ref, w3_ref, b3_ref, o_ref):
        t = _elu(jnp.dot(z_ref[...], w1_ref[...],
                         preferred_element_type=jnp.float32) + b1_ref[...])
        t = _elu(jnp.dot(t, w2_ref[...],
                         preferred_element_type=jnp.float32) + b2_ref[...])
        o_ref[...] = (jnp.dot(t, w3_ref[...],
                              preferred_element_type=jnp.float32) + b3_ref[...])

    const = lambda i: (0, 0)
    return pl.pallas_call(
        body,
        grid=(grid,),
        in_specs=[
            pl.BlockSpec((_CHUNK, D), lambda i: (i, 0)),
            pl.BlockSpec((D, H1), const), pl.BlockSpec((1, H1), const),
            pl.BlockSpec((H1, H2), const), pl.BlockSpec((1, H2), const),
            pl.BlockSpec((H2, F), const), pl.BlockSpec((1, F), const),
        ],
        out_specs=pl.BlockSpec((_CHUNK, F), lambda i: (i, 0)),
        out_shape=jax.ShapeDtypeStruct((N, F), jnp.float32),
    )(z, W1, b1.reshape(1, H1), W2, b2.reshape(1, H2), W3, b3.reshape(1, F))


def kernel(params, atom_types, is_scaffold, bond_info, block_ids, last_append_mask):
    n = atom_types.shape[0]
    begin = bond_info[:, 0]
    end = bond_info[:, 1]
    btype = bond_info[:, 2]

    at = jnp.where(is_scaffold == 1, atom_types + _NAT,
         jnp.where(last_append_mask == 1, atom_types + 2 * _NAT,
         jnp.where(last_append_mask == 2, atom_types + 3 * _NAT, atom_types)))
    at = jnp.where(is_scaffold == 1, at + _NAT, at)
    feats = jnp.take(params['emb'], at, axis=0)

    pad_e = _EPT_PAD - 25000
    ekey = jnp.pad((begin * _NBOND + btype).reshape(32, 25000),
                   ((0, 0), (0, pad_e)), constant_values=2**28).reshape(-1)
    eend = jnp.pad(end.reshape(32, 25000),
                   ((0, 0), (0, pad_e))).reshape(-1)

    for lp in params['layers']:
        bn = lp['bn']
        h = _bnl_matmul(feats, bn['gamma'], bn['beta'], bn['W'], bn['b'])
        parts = _edge_aggregate(h, ekey, eend)
        p0 = parts[0].reshape(_KEY_PAD // _NBOND, _NBOND * _BN_F)
        p1 = parts[1].reshape(_KEY_PAD // _NBOND, _NBOND * _BN_F)
        mlp = lp['mlp']
        W1 = mlp[0]['W']
        z = _mlp3_agg(h, p0, p1,
                      W1[:_BN_F], W1[_BN_F:_BN_F * (1 + _NBOND)], mlp[0]['b'],
                      mlp[1]['W'], mlp[1]['b'], mlp[2]['W'], mlp[2]['b'])
        feats = jnp.concatenate([feats, z], axis=-1)

    fin = params['final']
    out = _bnl_matmul(feats, fin['gamma'], fin['beta'], fin['W'], fin['b'])
    hp = _elu(out * params['pool_gamma'] + params['pool_beta'])

    seg_sum = jax.ops.segment_sum(hp, block_ids, num_segments=_N_BLOCKS)
    cnt = jax.ops.segment_sum(jnp.ones((n,), jnp.float32), block_ids,
                              num_segments=_N_BLOCKS)
    mol = seg_sum / jnp.maximum(cnt, 1.0)[:, None]

    ac = params['append_connect']
    D1 = out.shape[1]
    U = _bnl_matmul(out, ac['gamma'][:D1], ac['beta'][:D1], ac['W'][:D1], ac['b'])
    Vsmall = _elu(mol * ac['gamma'][D1:] + ac['beta'][D1:]) @ ac['W'][D1:]
    act_ac = U + jnp.take(Vsmall, block_ids, axis=0)

    ep = params['end']
    act_end = (_elu(mol * ep['gamma'] + ep['beta']) @ ep['W'] + ep['b'])[:, 0]

    row_max = jnp.max(act_ac, axis=-1)
    seg_max = jax.ops.segment_max(row_max, block_ids, num_segments=_N_BLOCKS)
    m = jnp.maximum(seg_max, act_end)
    ex = jnp.exp(act_ac - jnp.take(m, block_ids)[:, None])
    eb = jnp.exp(act_end - m)
    Z = jax.ops.segment_sum(jnp.sum(ex, axis=-1), block_ids,
                            num_segments=_N_BLOCKS) + eb
    p_ac = ex / jnp.take(Z, block_ids)[:, None]
    p_end = eb / Z
    p_append = p_ac[:, :_NAT * _NBOND].reshape(n, _NAT, _NBOND)
    p_connect = p_ac[:, _NAT * _NBOND:]
    return (p_append, p_connect, p_end)
